# baseline (speedup 1.0000x reference)
import functools

import jax
import jax.numpy as jnp
from jax import lax
from jax.experimental import pallas as pl
from jax.experimental.pallas import tpu as pltpu
from jax.experimental.pallas import tpu_sc as plsc

NC, NS, LANES = 2, 16, 16
CH = 48
SC_ROWS = 0
TC_BH = 8


def _sc_pos(h_part, w, d):
    nw = NC * NS
    assert h_part % nw == 0 and d % LANES == 0 and w % CH == 0
    rows_per_w = h_part // nw
    n_jc = w // CH
    assert n_jc & (n_jc - 1) == 0
    jc_shift = n_jc.bit_length() - 1
    jc_mask = n_jc - 1
    n_chunks = rows_per_w * n_jc
    assert n_chunks % 2 == 0
    nvec = d // LANES

    mesh = plsc.VectorSubcoreMesh(core_axis_name="c", subcore_axis_name="s")

    @functools.partial(
        pl.kernel,
        out_type=jax.ShapeDtypeStruct((h_part * w, d), jnp.float32),
        mesh=mesh,
        scratch_types=[
            pltpu.VMEM((w, d), jnp.float32),
            pltpu.VMEM((rows_per_w * d,), jnp.float32),
            pltpu.VMEM((CH, d), jnp.float32),
            pltpu.VMEM((CH, d), jnp.float32),
            pltpu.SemaphoreType.DMA,
            pltpu.SemaphoreType.DMA,
        ],
    )
    def body(wrow_hbm, wcol_hbm, out_hbm, wcol_v, wrow_v, buf0, buf1, sem0, sem1):
        wid = lax.axis_index("s") * NC + lax.axis_index("c")
        row0 = wid * rows_per_w
        pltpu.sync_copy(wcol_hbm, wcol_v)
        pltpu.sync_copy(wrow_hbm.at[pl.ds(row0 * d, rows_per_w * d)], wrow_v)

        bufs = (buf0, buf1)
        sems = (sem0, sem1)

        def out_off(k):
            return (row0 + (k >> jc_shift)) * w + (k & jc_mask) * CH

        def compute_chunk(k, buf):
            irel = k >> jc_shift
            jc = (k & jc_mask) * CH
            wrow_vecs = [
                wrow_v[pl.ds(irel * d + dv * LANES, LANES)] for dv in range(nvec)
            ]

            @plsc.parallel_loop(0, CH, unroll=8)
            def _jj_body(jj):
                row = jc + jj
                for dv in range(nvec):
                    sl = pl.ds(dv * LANES, LANES)
                    buf[jj, sl] = wcol_v[row, sl] + wrow_vecs[dv]

        def pair_body(t, c):
            for b in range(2):
                k = 2 * t + b

                @pl.when(t > 0)
                def _wait():
                    pltpu.make_async_copy(
                        bufs[b], out_hbm.at[pl.ds(out_off(k - 2), CH)], sems[b]
                    ).wait()

                compute_chunk(k, bufs[b])
                pltpu.async_copy(bufs[b], out_hbm.at[pl.ds(out_off(k), CH)], sems[b])
            return c

        lax.fori_loop(0, n_chunks // 2, pair_body, 0)
        for b in range(2):
            kp = n_chunks - 2 + b
            pltpu.make_async_copy(
                bufs[b], out_hbm.at[pl.ds(out_off(kp), CH)], sems[b]
            ).wait()

    return body


def _tc_body(w_row_ref, w_col_ref, out_ref):
    out_ref[...] = w_row_ref[...][:, None, :] + w_col_ref[...][None, :, :]


def _tc_pos(h_part, w, d):
    return pl.pallas_call(
        _tc_body,
        grid=(h_part // TC_BH,),
        in_specs=[
            pl.BlockSpec((TC_BH, d), lambda i: (i, 0)),
            pl.BlockSpec((w, d), lambda i: (0, 0)),
        ],
        out_specs=pl.BlockSpec((TC_BH, w, d), lambda i: (i, 0, 0)),
        out_shape=jax.ShapeDtypeStruct((h_part, w, d), jnp.float32),
    )


def kernel(input, W_row, W_col):
    h, w = input.shape[1], input.shape[2]
    d = W_row.shape[1]
    h_tc = h - SC_ROWS
    wcol = W_col[:w]
    tc_part = _tc_pos(h_tc, w, d)(W_row[:h_tc], wcol)
    if SC_ROWS == 0:
        return tc_part
    sc_part = _sc_pos(SC_ROWS, w, d)(W_row[h_tc:h].reshape(-1), wcol)
    return jnp.concatenate([tc_part, sc_part.reshape(SC_ROWS, w, d)], axis=0)


# speedup vs baseline: 1.0599x; 1.0599x over previous
import functools

import jax
import jax.numpy as jnp
from jax import lax
from jax.experimental import pallas as pl
from jax.experimental.pallas import tpu as pltpu
from jax.experimental.pallas import tpu_sc as plsc

NC, NS, LANES = 2, 16, 16
CH = 48
SC_ROWS = 0
TC_BH = 24


def _sc_pos(h_part, w, d):
    nw = NC * NS
    assert h_part % nw == 0 and d % LANES == 0 and w % CH == 0
    rows_per_w = h_part // nw
    n_jc = w // CH
    assert n_jc & (n_jc - 1) == 0
    jc_shift = n_jc.bit_length() - 1
    jc_mask = n_jc - 1
    n_chunks = rows_per_w * n_jc
    assert n_chunks % 2 == 0
    nvec = d // LANES

    mesh = plsc.VectorSubcoreMesh(core_axis_name="c", subcore_axis_name="s")

    @functools.partial(
        pl.kernel,
        out_type=jax.ShapeDtypeStruct((h_part * w, d), jnp.float32),
        mesh=mesh,
        scratch_types=[
            pltpu.VMEM((w, d), jnp.float32),
            pltpu.VMEM((rows_per_w * d,), jnp.float32),
            pltpu.VMEM((CH, d), jnp.float32),
            pltpu.VMEM((CH, d), jnp.float32),
            pltpu.SemaphoreType.DMA,
            pltpu.SemaphoreType.DMA,
        ],
    )
    def body(wrow_hbm, wcol_hbm, out_hbm, wcol_v, wrow_v, buf0, buf1, sem0, sem1):
        wid = lax.axis_index("s") * NC + lax.axis_index("c")
        row0 = wid * rows_per_w
        pltpu.sync_copy(wcol_hbm, wcol_v)
        pltpu.sync_copy(wrow_hbm.at[pl.ds(row0 * d, rows_per_w * d)], wrow_v)

        bufs = (buf0, buf1)
        sems = (sem0, sem1)

        def out_off(k):
            return (row0 + (k >> jc_shift)) * w + (k & jc_mask) * CH

        def compute_chunk(k, buf):
            irel = k >> jc_shift
            jc = (k & jc_mask) * CH
            wrow_vecs = [
                wrow_v[pl.ds(irel * d + dv * LANES, LANES)] for dv in range(nvec)
            ]

            @plsc.parallel_loop(0, CH, unroll=8)
            def _jj_body(jj):
                row = jc + jj
                for dv in range(nvec):
                    sl = pl.ds(dv * LANES, LANES)
                    buf[jj, sl] = wcol_v[row, sl] + wrow_vecs[dv]

        def pair_body(t, c):
            for b in range(2):
                k = 2 * t + b

                @pl.when(t > 0)
                def _wait():
                    pltpu.make_async_copy(
                        bufs[b], out_hbm.at[pl.ds(out_off(k - 2), CH)], sems[b]
                    ).wait()

                compute_chunk(k, bufs[b])
                pltpu.async_copy(bufs[b], out_hbm.at[pl.ds(out_off(k), CH)], sems[b])
            return c

        lax.fori_loop(0, n_chunks // 2, pair_body, 0)
        for b in range(2):
            kp = n_chunks - 2 + b
            pltpu.make_async_copy(
                bufs[b], out_hbm.at[pl.ds(out_off(kp), CH)], sems[b]
            ).wait()

    return body


def _tc_body(w_row_ref, w_col_ref, out_ref):
    out_ref[...] = w_row_ref[...][:, None, :] + w_col_ref[...][None, :, :]


def _tc_pos(h_part, w, d):
    return pl.pallas_call(
        _tc_body,
        grid=(h_part // TC_BH,),
        in_specs=[
            pl.BlockSpec((TC_BH, d), lambda i: (i, 0)),
            pl.BlockSpec((w, d), lambda i: (0, 0)),
        ],
        out_specs=pl.BlockSpec((TC_BH, w, d), lambda i: (i, 0, 0)),
        out_shape=jax.ShapeDtypeStruct((h_part, w, d), jnp.float32),
    )


def kernel(input, W_row, W_col):
    h, w = input.shape[1], input.shape[2]
    d = W_row.shape[1]
    h_tc = h - SC_ROWS
    wcol = W_col[:w]
    tc_part = _tc_pos(h_tc, w, d)(W_row[:h_tc], wcol)
    if SC_ROWS == 0:
        return tc_part
    sc_part = _sc_pos(SC_ROWS, w, d)(W_row[h_tc:h].reshape(-1), wcol)
    return jnp.concatenate([tc_part, sc_part.reshape(SC_ROWS, w, d)], axis=0)


# speedup vs baseline: 1.0736x; 1.0129x over previous
import jax
import jax.numpy as jnp
from jax.experimental import pallas as pl

TC_BH = 16


def _pos_body(w_row_ref, w_col_ref, out_ref):
    out_ref[...] = w_row_ref[...][:, None, :] + w_col_ref[...][None, :, :]


def kernel(input, W_row, W_col):
    h, w = input.shape[1], input.shape[2]
    d = W_row.shape[1]
    return pl.pallas_call(
        _pos_body,
        grid=(h // TC_BH,),
        in_specs=[
            pl.BlockSpec((TC_BH, d), lambda i: (i, 0)),
            pl.BlockSpec((w, d), lambda i: (0, 0)),
        ],
        out_specs=pl.BlockSpec((TC_BH, w, d), lambda i: (i, 0, 0)),
        out_shape=jax.ShapeDtypeStruct((h, w, d), jnp.float32),
    )(W_row[:h], W_col[:w])
